# baseline (speedup 1.0000x reference)
import jax
import jax.numpy as jnp
from jax.experimental import pallas as pl
from jax.experimental.pallas import tpu as pltpu

_MIN_NORM = 1e-15
_BALL_EPS = 1e-5


def _rownorm(v):
    return jnp.maximum(jnp.sqrt(jnp.sum(v * v, axis=-1, keepdims=True)), _MIN_NORM)


def _fused_body(x_ref, adj_ref, w_ref, b_ref, out_ref, s_ref):
    i = pl.program_id(0)
    maxnorm = 1.0 - _BALL_EPS
    d_in = x_ref.shape[1]

    @pl.when(i == 0)
    def _prologue():
        xv = x_ref[...]
        norm = _rownorm(xv)
        n2 = jnp.minimum(norm, maxnorm)
        at = 0.5 * jnp.log((1.0 + n2) / (1.0 - n2))
        s_ref[:, :d_in] = (xv * (at / norm)).astype(jnp.bfloat16)
        s_ref[:, d_in:] = jnp.ones_like(s_ref[:, d_in:])

    acc2 = jax.lax.dot_general(
        adj_ref[...].astype(jnp.bfloat16), s_ref[...],
        (((1,), (0,)), ((), ())),
        preferred_element_type=jnp.float32,
    )
    y = acc2[:, :d_in]
    rs = acc2[:, d_in:d_in + 1]
    acc = jax.lax.dot_general(
        y, w_ref[...], (((1,), (0,)), ((), ())),
        preferred_element_type=jnp.float32,
    ) + rs * b_ref[...]
    norm = _rownorm(acc)
    t = jnp.tanh(norm)
    out_ref[...] = acc * (jnp.minimum(t, maxnorm) / norm)


def kernel(x, adj, weight, bias):
    n, d_in = x.shape
    d_out = weight.shape[1]
    bias2 = bias.reshape(1, d_out).astype(jnp.float32)

    bm = 400 if n % 400 == 0 else n
    out = pl.pallas_call(
        _fused_body,
        grid=(n // bm,),
        in_specs=[
            pl.BlockSpec((n, d_in), lambda i: (0, 0)),
            pl.BlockSpec((bm, n), lambda i: (i, 0)),
            pl.BlockSpec((d_in, d_out), lambda i: (0, 0)),
            pl.BlockSpec((1, d_out), lambda i: (0, 0)),
        ],
        out_specs=pl.BlockSpec((bm, d_out), lambda i: (i, 0)),
        out_shape=jax.ShapeDtypeStruct((n, d_out), jnp.float32),
        scratch_shapes=[pltpu.VMEM((n, d_in + 8), jnp.bfloat16)],
        compiler_params=pltpu.CompilerParams(
            dimension_semantics=("arbitrary",)),
    )(x, adj, weight, bias2)
    return out
